# baseline (speedup 1.0000x reference)
import functools

import jax
import jax.numpy as jnp
from jax import lax
from jax.experimental import pallas as pl
from jax.experimental.pallas import tpu as pltpu
from jax.experimental.pallas import tpu_sc as plsc

_N = 10000
_D = 128
_H = 2
_E = 320000

_N_PAD = 10240
_NW = 32
_EPT = 10240
_E_PAD = _NW * _EPT
_CH1 = 80
_CH2 = 160
_TPW = _N_PAD // 16

_mesh = plsc.VectorSubcoreMesh(core_axis_name="c", subcore_axis_name="s")


@functools.partial(
    pl.kernel,
    out_type=jax.ShapeDtypeStruct((2 * _N_PAD,), jnp.float32),
    mesh=_mesh,
    scratch_types=[
        pltpu.VMEM((_CH1, 128), jnp.int32),
        pltpu.VMEM((128,), jnp.float32),
        pltpu.VMEM_SHARED((_N_PAD,), jnp.float32),
        pltpu.SemaphoreType.DMA,
    ],
)
def _sc_degree(dstr_hbm, zeros_hbm, out_hbm, dst_v, ones_v, cnt_sp, ssem):
    cid = lax.axis_index("c")
    sid = lax.axis_index("s")
    wid = sid * 2 + cid
    r0 = sid * _TPW
    pltpu.sync_copy(zeros_hbm.at[pl.ds(r0, _TPW)], cnt_sp.at[pl.ds(r0, _TPW)])
    pltpu.sync_copy(dstr_hbm.at[wid], dst_v)
    for k in range(8):
        ones_v[pl.ds(k * 16, 16)] = jnp.ones((16,), jnp.float32)
    plsc.subcore_barrier()

    def fire(j, carry):
        pltpu.async_copy(ones_v, cnt_sp.at[dst_v.at[j]], ssem, add=True)
        return carry

    def drain(j, carry):
        pltpu.make_async_copy(ones_v, cnt_sp.at[dst_v.at[0]], ssem).wait()
        return carry

    lax.fori_loop(0, _CH1, fire, 0)
    lax.fori_loop(0, _CH1, drain, 0)
    plsc.subcore_barrier()
    pltpu.sync_copy(cnt_sp.at[pl.ds(r0, _TPW)],
                    out_hbm.at[pl.ds(cid * _N_PAD + r0, _TPW)])


@functools.partial(
    pl.kernel,
    out_type=jax.ShapeDtypeStruct((4 * _N_PAD,), jnp.float32),
    mesh=_mesh,
    scratch_types=[
        pltpu.VMEM((_CH2, 128), jnp.int32),
        pltpu.VMEM((_CH2, 128), jnp.int32),
        pltpu.VMEM((8, 128), jnp.float32),
        pltpu.VMEM_SHARED((2 * _N_PAD,), jnp.float32),
    ] + [pltpu.SemaphoreType.DMA] * 16,
)
def _sc_edge_agg(y_hbm, src2_hbm, dst2_hbm, zeros_hbm, out_hbm,
                 src_v, dst_v, upd_v, z_sp, *sems):
    cid = lax.axis_index("c")
    sid = lax.axis_index("s")
    wid = sid * 2 + cid
    f0 = sid * (2 * _TPW)
    gsems, ssems = sems[:8], sems[8:]
    pltpu.sync_copy(zeros_hbm.at[pl.ds(f0, 2 * _TPW)], z_sp.at[pl.ds(f0, 2 * _TPW)])
    pltpu.sync_copy(src2_hbm.at[wid], src_v)
    pltpu.sync_copy(dst2_hbm.at[wid], dst_v)
    plsc.subcore_barrier()

    def fire_g(c, b):
        pltpu.async_copy(y_hbm.at[src_v.at[c]], upd_v.at[b], gsems[b])

    def drain_g(b):
        pltpu.make_async_copy(y_hbm.at[src_v.at[0]], upd_v.at[b],
                              gsems[b]).wait()

    def fire_s(c, b):
        pltpu.async_copy(upd_v.at[b], z_sp.at[dst_v.at[c]], ssems[b],
                         add=True)

    def drain_s(b):
        pltpu.make_async_copy(upd_v.at[b], z_sp.at[dst_v.at[0]],
                              ssems[b]).wait()

    for b in range(4):
        fire_g(b, b)
    for c in range(4):
        drain_g(c)
        fire_s(c, c)
        fire_g(c + 4, (c + 4) % 8)

    def body(i, carry):
        c0 = 4 + i * 8
        for k in range(8):
            c = c0 + k
            b = (4 + k) % 8
            drain_g(b)
            fire_s(c, b)
            drain_s((b + 4) % 8)
            fire_g(c + 4, (b + 4) % 8)
        return carry

    lax.fori_loop(0, (_CH2 - 8) // 8, body, 0)
    for k in range(4):
        c = _CH2 - 4 + k
        b = c % 8
        drain_g(b)
        fire_s(c, b)
        drain_s((b + 4) % 8)
    for k in range(4):
        drain_s((_CH2 - 4 + k) % 8)
    plsc.subcore_barrier()
    pltpu.sync_copy(z_sp.at[pl.ds(f0, 2 * _TPW)],
                    out_hbm.at[pl.ds(cid * 2 * _N_PAD + f0, 2 * _TPW)])


_BLK = 1000


def _tc1_body(x_ref, prm_ref, d0_ref, d1_ref, y1_ref, dinv_ref):
    prm = prm_ref[...]
    xb = x_ref[...]
    dinv = lax.rsqrt(d0_ref[...] + d1_ref[...] + 1.0)
    h0 = jnp.sum(xb * prm[0:1, :], axis=1, keepdims=True)
    h1 = jnp.sum(xb * prm[1:2, :], axis=1, keepdims=True)
    y1_ref[...] = jnp.concatenate([h0, h1], axis=1) * dinv
    dinv_ref[...] = dinv


_tc1 = pl.pallas_call(
    _tc1_body,
    grid=(_N // _BLK,),
    in_specs=[
        pl.BlockSpec((_BLK, _D), lambda i: (i, 0)),
        pl.BlockSpec((8, _D), lambda i: (0, 0)),
        pl.BlockSpec((_BLK, 1), lambda i: (i, 0)),
        pl.BlockSpec((_BLK, 1), lambda i: (i, 0)),
    ],
    out_specs=[
        pl.BlockSpec((_BLK, _H), lambda i: (i, 0)),
        pl.BlockSpec((_BLK, 1), lambda i: (i, 0)),
    ],
    out_shape=[
        jax.ShapeDtypeStruct((_N, _H), jnp.float32),
        jax.ShapeDtypeStruct((_N, 1), jnp.float32),
    ],
)


def _tc2_body(z1a_ref, z1b_ref, y1_ref, dinv_ref, prm_ref, y2_ref):
    prm = prm_ref[...]
    g = prm[5:6, 0:_H]
    bln = prm[6:7, 0:_H]
    b1r = prm[7:8, 0:_H]
    dinv = dinv_ref[...]
    out1 = dinv * (z1a_ref[...] + z1b_ref[...] + y1_ref[...]) + b1r
    mu = jnp.mean(out1, axis=1, keepdims=True)
    var = jnp.mean((out1 - mu) ** 2, axis=1, keepdims=True)
    a = (out1 - mu) * lax.rsqrt(var + 1e-5) * g + bln
    y2_ref[...] = jnp.maximum(a, 0.0) * dinv


_tc2 = pl.pallas_call(
    _tc2_body,
    grid=(_N // _BLK,),
    in_specs=[
        pl.BlockSpec((_BLK, _H), lambda i: (i, 0)),
        pl.BlockSpec((_BLK, _H), lambda i: (i, 0)),
        pl.BlockSpec((_BLK, _H), lambda i: (i, 0)),
        pl.BlockSpec((_BLK, 1), lambda i: (i, 0)),
        pl.BlockSpec((8, _D), lambda i: (0, 0)),
    ],
    out_specs=pl.BlockSpec((_BLK, _H), lambda i: (i, 0)),
    out_shape=jax.ShapeDtypeStruct((_N, _H), jnp.float32),
)


def _tc3_body(z2a_ref, z2b_ref, y2_ref, dinv_ref, prm_ref, sf_ref, o_ref):
    prm = prm_ref[...]
    p = dinv_ref[...] * (z2a_ref[...] + z2b_ref[...] + y2_ref[...])
    h = p[:, 0:1] * prm[2:3, :] + p[:, 1:2] * prm[3:4, :] + prm[4:5, :]
    o_ref[...] = jnp.maximum(h, 0.0) * sf_ref[...]


_tc3 = pl.pallas_call(
    _tc3_body,
    grid=(_N // _BLK,),
    in_specs=[
        pl.BlockSpec((_BLK, _H), lambda i: (i, 0)),
        pl.BlockSpec((_BLK, _H), lambda i: (i, 0)),
        pl.BlockSpec((_BLK, _H), lambda i: (i, 0)),
        pl.BlockSpec((_BLK, 1), lambda i: (i, 0)),
        pl.BlockSpec((8, _D), lambda i: (0, 0)),
        pl.BlockSpec((_BLK, 1), lambda i: (i, 0)),
    ],
    out_specs=pl.BlockSpec((_BLK, _D), lambda i: (i, 0)),
    out_shape=jax.ShapeDtypeStruct((_N, _D), jnp.float32),
)


def kernel(x, edge_index, size_factors, W1, b1, ln_g, ln_b, W2, b2):
    src, dst = edge_index[0], edge_index[1]

    padn = _E_PAD - _E
    pad_idx = _N + (jnp.arange(padn, dtype=jnp.int32) % 128)
    src_p = jnp.concatenate([src, pad_idx])
    dst_p = jnp.concatenate([dst, pad_idx])
    dstr = dst_p.reshape(_NW, _CH1, 128)
    src2 = (src_p[:, None] * 2 + jnp.arange(2, dtype=jnp.int32)
            ).reshape(_NW, _CH2, 128)
    dst2 = (dst_p[:, None] * 2 + jnp.arange(2, dtype=jnp.int32)
            ).reshape(_NW, _CH2, 128)

    zeros = jnp.zeros((2 * _N_PAD,), jnp.float32)

    def pad128(v):
        return jnp.zeros((_D,), jnp.float32).at[: v.shape[0]].set(v)

    prm = jnp.stack([
        W1[:, 0], W1[:, 1],
        W2[0], W2[1],
        b2,
        pad128(ln_g), pad128(ln_b), pad128(b1),
    ])

    hist = _sc_degree(dstr, zeros[:_N_PAD])
    deg0 = hist[:_N][:, None]
    deg1 = hist[_N_PAD:_N_PAD + _N][:, None]

    y1, dinv = _tc1(x, prm, deg0, deg1)

    pad_rows = jnp.zeros((_N_PAD - _N, _H), jnp.float32)

    def agg(y):
        y_flat = jnp.concatenate([y, pad_rows]).reshape(-1)
        z = _sc_edge_agg(y_flat, src2, dst2, zeros)
        za = z[: 2 * _N_PAD].reshape(_N_PAD, _H)[:_N]
        zb = z[2 * _N_PAD:].reshape(_N_PAD, _H)[:_N]
        return za, zb

    z1a, z1b = agg(y1)
    y2 = _tc2(z1a, z1b, y1, dinv, prm)

    z2a, z2b = agg(y2)
    return _tc3(z2a, z2b, y2, dinv, prm, size_factors)


# speedup vs baseline: 1.3022x; 1.3022x over previous
import functools

import jax
import jax.numpy as jnp
from jax import lax
from jax.experimental import pallas as pl
from jax.experimental.pallas import tpu as pltpu
from jax.experimental.pallas import tpu_sc as plsc

_N = 10000
_D = 128
_H = 2
_E = 320000

_N_PAD = 10240
_NW = 32
_EPT = 10240
_E_PAD = _NW * _EPT
_CH1 = 80
_CH2 = 160
_TPW = _N_PAD // 16

_mesh = plsc.VectorSubcoreMesh(core_axis_name="c", subcore_axis_name="s")


@functools.partial(
    pl.kernel,
    out_type=jax.ShapeDtypeStruct((2 * _N_PAD,), jnp.float32),
    mesh=_mesh,
    scratch_types=[
        pltpu.VMEM((_CH1, 128), jnp.int32),
        pltpu.VMEM((128,), jnp.float32),
        pltpu.VMEM_SHARED((_N_PAD,), jnp.float32),
        pltpu.SemaphoreType.DMA,
    ],
)
def _sc_degree(dstr_hbm, zeros_hbm, out_hbm, dst_v, ones_v, cnt_sp, ssem):
    cid = lax.axis_index("c")
    sid = lax.axis_index("s")
    wid = sid * 2 + cid
    r0 = sid * _TPW
    pltpu.sync_copy(zeros_hbm.at[pl.ds(r0, _TPW)], cnt_sp.at[pl.ds(r0, _TPW)])
    pltpu.sync_copy(dstr_hbm.at[wid], dst_v)
    for k in range(8):
        ones_v[pl.ds(k * 16, 16)] = jnp.ones((16,), jnp.float32)
    plsc.subcore_barrier()

    def fire(j, carry):
        pltpu.async_copy(ones_v, cnt_sp.at[dst_v.at[j]], ssem, add=True)
        return carry

    def drain(j, carry):
        pltpu.make_async_copy(ones_v, cnt_sp.at[dst_v.at[0]], ssem).wait()
        return carry

    lax.fori_loop(0, _CH1, fire, 0)
    lax.fori_loop(0, _CH1, drain, 0)
    plsc.subcore_barrier()
    pltpu.sync_copy(cnt_sp.at[pl.ds(r0, _TPW)],
                    out_hbm.at[pl.ds(cid * _N_PAD + r0, _TPW)])


@functools.partial(
    pl.kernel,
    out_type=jax.ShapeDtypeStruct((4 * _N_PAD,), jnp.float32),
    mesh=_mesh,
    scratch_types=[
        pltpu.VMEM((_CH2, 128), jnp.int32),
        pltpu.VMEM((_CH2, 128), jnp.int32),
        pltpu.VMEM((8, 128), jnp.float32),
        pltpu.VMEM_SHARED((2 * _N_PAD,), jnp.float32),
        pltpu.VMEM_SHARED((2 * _N_PAD,), jnp.float32),
    ] + [pltpu.SemaphoreType.DMA] * 16,
)
def _sc_edge_agg(y_hbm, src2_hbm, dst2_hbm, zeros_hbm, out_hbm,
                 src_v, dst_v, upd_v, y_sp, z_sp, *sems):
    cid = lax.axis_index("c")
    sid = lax.axis_index("s")
    wid = sid * 2 + cid
    f0 = sid * (2 * _TPW)
    gsems, ssems = sems[:8], sems[8:]
    pltpu.sync_copy(zeros_hbm.at[pl.ds(f0, 2 * _TPW)], z_sp.at[pl.ds(f0, 2 * _TPW)])
    pltpu.sync_copy(y_hbm.at[pl.ds(f0, 2 * _TPW)], y_sp.at[pl.ds(f0, 2 * _TPW)])
    pltpu.sync_copy(src2_hbm.at[wid], src_v)
    pltpu.sync_copy(dst2_hbm.at[wid], dst_v)
    plsc.subcore_barrier()

    def fire_g(c, b):
        pltpu.async_copy(y_sp.at[src_v.at[c]], upd_v.at[b], gsems[b])

    def drain_g(b):
        pltpu.make_async_copy(y_sp.at[src_v.at[0]], upd_v.at[b],
                              gsems[b]).wait()

    nch = _CH2

    def fire_s(c, b):
        pltpu.async_copy(upd_v.at[b], z_sp.at[dst_v.at[c]], ssems[b],
                         add=True)

    def drain_s(b):
        pltpu.make_async_copy(upd_v.at[b], z_sp.at[dst_v.at[0]],
                              ssems[b]).wait()

    for b in range(4):
        fire_g(b, b)
    for c in range(4):
        drain_g(c)
        fire_s(c, c)
        fire_g(c + 4, (c + 4) % 8)

    def body(i, carry):
        c0 = 4 + i * 8
        for k in range(8):
            c = c0 + k
            b = (4 + k) % 8
            drain_g(b)
            fire_s(c, b)
            drain_s((b + 4) % 8)
            fire_g(c + 4, (b + 4) % 8)
        return carry

    lax.fori_loop(0, (nch - 8) // 8, body, 0)
    for k in range(4):
        c = nch - 4 + k
        b = c % 8
        drain_g(b)
        fire_s(c, b)
        drain_s((b + 4) % 8)
    for k in range(4):
        drain_s((nch - 4 + k) % 8)
    plsc.subcore_barrier()
    pltpu.sync_copy(z_sp.at[pl.ds(f0, 2 * _TPW)],
                    out_hbm.at[pl.ds(cid * 2 * _N_PAD + f0, 2 * _TPW)])


_BLK = 1000


def _tc1_body(x_ref, prm_ref, d0_ref, d1_ref, y1_ref, dinv_ref):
    prm = prm_ref[...]
    xb = x_ref[...]
    dinv = lax.rsqrt(d0_ref[...] + d1_ref[...] + 1.0)
    h0 = jnp.sum(xb * prm[0:1, :], axis=1, keepdims=True)
    h1 = jnp.sum(xb * prm[1:2, :], axis=1, keepdims=True)
    y1_ref[...] = jnp.concatenate([h0, h1], axis=1) * dinv
    dinv_ref[...] = dinv


_tc1 = pl.pallas_call(
    _tc1_body,
    grid=(_N // _BLK,),
    in_specs=[
        pl.BlockSpec((_BLK, _D), lambda i: (i, 0)),
        pl.BlockSpec((8, _D), lambda i: (0, 0)),
        pl.BlockSpec((_BLK, 1), lambda i: (i, 0)),
        pl.BlockSpec((_BLK, 1), lambda i: (i, 0)),
    ],
    out_specs=[
        pl.BlockSpec((_BLK, _H), lambda i: (i, 0)),
        pl.BlockSpec((_BLK, 1), lambda i: (i, 0)),
    ],
    out_shape=[
        jax.ShapeDtypeStruct((_N, _H), jnp.float32),
        jax.ShapeDtypeStruct((_N, 1), jnp.float32),
    ],
)


def _tc2_body(z1a_ref, z1b_ref, y1_ref, dinv_ref, prm_ref, y2_ref):
    prm = prm_ref[...]
    g = prm[5:6, 0:_H]
    bln = prm[6:7, 0:_H]
    b1r = prm[7:8, 0:_H]
    dinv = dinv_ref[...]
    out1 = dinv * (z1a_ref[...] + z1b_ref[...] + y1_ref[...]) + b1r
    mu = jnp.mean(out1, axis=1, keepdims=True)
    var = jnp.mean((out1 - mu) ** 2, axis=1, keepdims=True)
    a = (out1 - mu) * lax.rsqrt(var + 1e-5) * g + bln
    y2_ref[...] = jnp.maximum(a, 0.0) * dinv


_tc2 = pl.pallas_call(
    _tc2_body,
    grid=(_N // _BLK,),
    in_specs=[
        pl.BlockSpec((_BLK, _H), lambda i: (i, 0)),
        pl.BlockSpec((_BLK, _H), lambda i: (i, 0)),
        pl.BlockSpec((_BLK, _H), lambda i: (i, 0)),
        pl.BlockSpec((_BLK, 1), lambda i: (i, 0)),
        pl.BlockSpec((8, _D), lambda i: (0, 0)),
    ],
    out_specs=pl.BlockSpec((_BLK, _H), lambda i: (i, 0)),
    out_shape=jax.ShapeDtypeStruct((_N, _H), jnp.float32),
)


def _tc3_body(z2a_ref, z2b_ref, y2_ref, dinv_ref, prm_ref, sf_ref, o_ref):
    prm = prm_ref[...]
    p = dinv_ref[...] * (z2a_ref[...] + z2b_ref[...] + y2_ref[...])
    h = p[:, 0:1] * prm[2:3, :] + p[:, 1:2] * prm[3:4, :] + prm[4:5, :]
    o_ref[...] = jnp.maximum(h, 0.0) * sf_ref[...]


_tc3 = pl.pallas_call(
    _tc3_body,
    grid=(_N // _BLK,),
    in_specs=[
        pl.BlockSpec((_BLK, _H), lambda i: (i, 0)),
        pl.BlockSpec((_BLK, _H), lambda i: (i, 0)),
        pl.BlockSpec((_BLK, _H), lambda i: (i, 0)),
        pl.BlockSpec((_BLK, 1), lambda i: (i, 0)),
        pl.BlockSpec((8, _D), lambda i: (0, 0)),
        pl.BlockSpec((_BLK, 1), lambda i: (i, 0)),
    ],
    out_specs=pl.BlockSpec((_BLK, _D), lambda i: (i, 0)),
    out_shape=jax.ShapeDtypeStruct((_N, _D), jnp.float32),
)


def kernel(x, edge_index, size_factors, W1, b1, ln_g, ln_b, W2, b2):
    src, dst = edge_index[0], edge_index[1]

    padn = _E_PAD - _E
    pad_idx = _N + (jnp.arange(padn, dtype=jnp.int32) % 128)
    src_p = jnp.concatenate([src, pad_idx])
    dst_p = jnp.concatenate([dst, pad_idx])
    dstr = dst_p.reshape(_NW, _CH1, 128)
    two = jnp.arange(2, dtype=jnp.int32)
    src2 = (src_p[:, None] * 2 + two).reshape(_NW, _CH2, 128)
    dst2 = (dst_p[:, None] * 2 + two).reshape(_NW, _CH2, 128)

    zeros1 = jnp.zeros((_N_PAD,), jnp.float32)
    zeros2 = jnp.zeros((2 * _N_PAD,), jnp.float32)

    def pad128(v):
        return jnp.zeros((_D,), jnp.float32).at[: v.shape[0]].set(v)

    prm = jnp.stack([
        W1[:, 0], W1[:, 1],
        W2[0], W2[1],
        b2,
        pad128(ln_g), pad128(ln_b), pad128(b1),
    ])

    hist = _sc_degree(dstr, zeros1)
    deg0 = hist[:_N][:, None]
    deg1 = hist[_N_PAD:_N_PAD + _N][:, None]

    y1, dinv = _tc1(x, prm, deg0, deg1)

    pad_rows = jnp.zeros((_N_PAD - _N, _H), jnp.float32)

    def agg(y):
        y_flat = jnp.concatenate([y, pad_rows]).reshape(-1)
        z = _sc_edge_agg(y_flat, src2, dst2, zeros2)
        za = z[: 2 * _N_PAD].reshape(_N_PAD, _H)[:_N]
        zb = z[2 * _N_PAD:].reshape(_N_PAD, _H)[:_N]
        return za, zb

    z1a, z1b = agg(y1)
    y2 = _tc2(z1a, z1b, y1, dinv, prm)

    z2a, z2b = agg(y2)
    return _tc3(z2a, z2b, y2, dinv, prm, size_factors)


# speedup vs baseline: 1.8008x; 1.3829x over previous
import functools

import jax
import jax.numpy as jnp
from jax import lax
from jax.experimental import pallas as pl
from jax.experimental.pallas import tpu as pltpu
from jax.experimental.pallas import tpu_sc as plsc

_N = 10000
_D = 128
_H = 2
_E = 320000

_N_PAD = 10240
_NW = 32
_EPT = 10240
_E_PAD = _NW * _EPT
_CHH = _E_PAD // (128 * 16)
_CH2 = 2 * _EPT // 128
_TPW = _N_PAD // 16

_mesh = plsc.VectorSubcoreMesh(core_axis_name="c", subcore_axis_name="s")


def _rsqrt16(x):
    i = lax.bitcast_convert_type(x, jnp.int32)
    r = lax.bitcast_convert_type(jnp.int32(0x5F3759DF) - (i >> 1),
                                 jnp.float32)
    for _ in range(3):
        r = r * (1.5 - 0.5 * x * r * r)
    return r


def _edge_agg_loop(y_sp, z_sp, src_v, dst_v, upd_v, gsems, ssems):
    def fire_g(c, b):
        pltpu.async_copy(y_sp.at[src_v.at[c]], upd_v.at[b], gsems[b])

    def drain_g(b):
        pltpu.make_async_copy(y_sp.at[src_v.at[0]], upd_v.at[b],
                              gsems[b]).wait()

    def fire_s(c, b):
        pltpu.async_copy(upd_v.at[b], z_sp.at[dst_v.at[c]], ssems[b],
                         add=True)

    def drain_s(b):
        pltpu.make_async_copy(upd_v.at[b], z_sp.at[dst_v.at[0]],
                              ssems[b]).wait()

    for b in range(4):
        fire_g(b, b)
    for c in range(4):
        drain_g(c)
        fire_s(c, c)
        fire_g(c + 4, (c + 4) % 8)

    def body(i, carry):
        c0 = 4 + i * 8
        for k in range(8):
            c = c0 + k
            b = (4 + k) % 8
            drain_g(b)
            fire_s(c, b)
            drain_s((b + 4) % 8)
            fire_g(c + 4, (b + 4) % 8)
        return carry

    lax.fori_loop(0, (_CH2 - 8) // 8, body, 0)
    for k in range(4):
        c = _CH2 - 4 + k
        b = c % 8
        drain_g(b)
        fire_s(c, b)
        drain_s((b + 4) % 8)
    for k in range(4):
        drain_s((_CH2 - 4 + k) % 8)


@functools.partial(
    pl.kernel,
    out_type=[
        jax.ShapeDtypeStruct((_N_PAD,), jnp.float32),
        jax.ShapeDtypeStruct((2 * _N_PAD,), jnp.float32),
        jax.ShapeDtypeStruct((4 * _N_PAD,), jnp.float32),
    ],
    mesh=_mesh,
    scratch_types=[
        pltpu.VMEM((_CHH, 128), jnp.int32),
        pltpu.VMEM((_CH2, 128), jnp.int32),
        pltpu.VMEM((_CH2, 128), jnp.int32),
        pltpu.VMEM((128,), jnp.float32),
        pltpu.VMEM((8, 128), jnp.float32),
        pltpu.VMEM((_TPW,), jnp.float32),
        pltpu.VMEM((_TPW,), jnp.float32),
        pltpu.VMEM((_TPW,), jnp.float32),
        pltpu.VMEM((_TPW,), jnp.float32),
        pltpu.VMEM((_TPW,), jnp.float32),
        pltpu.VMEM_SHARED((_N_PAD,), jnp.float32),
        pltpu.VMEM_SHARED((2 * _N_PAD,), jnp.float32),
        pltpu.VMEM_SHARED((2 * _N_PAD,), jnp.float32),
    ] + [pltpu.SemaphoreType.DMA] * 17,
)
def _sc_layer1(h1p_hbm, dsth_hbm, src2_hbm, dst2_hbm, zeros_hbm,
               deg_hbm, y1_hbm, z_hbm,
               dsth_v, src_v, dst_v, ones_v, upd_v,
               degb, h0b, h1b, yb0, yb1,
               cnt_sp, y_sp, z_sp, *sems):
    cid = lax.axis_index("c")
    sid = lax.axis_index("s")
    wid = sid * 2 + cid
    r0 = sid * _TPW
    f0 = sid * (2 * _TPW)
    gsems, ssems, hsem = sems[:8], sems[8:16], sems[16]

    pltpu.sync_copy(zeros_hbm.at[pl.ds(r0, _TPW)], cnt_sp.at[pl.ds(r0, _TPW)])
    pltpu.sync_copy(dsth_hbm.at[pl.ds(sid * _CHH, _CHH)], dsth_v)
    for k in range(8):
        ones_v[pl.ds(k * 16, 16)] = jnp.ones((16,), jnp.float32)
    plsc.subcore_barrier()

    def fire_h(j, carry):
        pltpu.async_copy(ones_v, cnt_sp.at[dsth_v.at[j]], hsem, add=True)
        return carry

    def drain_h(j, carry):
        pltpu.make_async_copy(ones_v, cnt_sp.at[dsth_v.at[0]], hsem).wait()
        return carry

    lax.fori_loop(0, _CHH, fire_h, 0)
    pltpu.sync_copy(src2_hbm.at[wid], src_v)
    pltpu.sync_copy(dst2_hbm.at[wid], dst_v)
    pltpu.sync_copy(h1p_hbm.at[pl.ds(r0, _TPW)], h0b)
    pltpu.sync_copy(h1p_hbm.at[pl.ds(_N_PAD + r0, _TPW)], h1b)
    pltpu.sync_copy(zeros_hbm.at[pl.ds(f0, 2 * _TPW)],
                    z_sp.at[pl.ds(f0, 2 * _TPW)])
    lax.fori_loop(0, _CHH, drain_h, 0)
    plsc.subcore_barrier()

    pltpu.sync_copy(cnt_sp.at[pl.ds(r0, _TPW)], degb)
    for v in range(_TPW // 16):
        ix = pl.ds(v * 16, 16)
        dinv = _rsqrt16(degb[ix] + 1.0)
        yb0[ix] = h0b[ix] * dinv
        yb1[ix] = h1b[ix] * dinv
    pltpu.sync_copy(yb0, y_sp.at[pl.ds(r0, _TPW)])
    pltpu.sync_copy(yb1, y_sp.at[pl.ds(_N_PAD + r0, _TPW)])
    @pl.when(cid == 0)
    def _():
        pltpu.sync_copy(cnt_sp.at[pl.ds(r0, _TPW)],
                        deg_hbm.at[pl.ds(r0, _TPW)])
    plsc.subcore_barrier()
    pltpu.sync_copy(y_sp.at[pl.ds(wid * _TPW, _TPW)],
                    y1_hbm.at[pl.ds(wid * _TPW, _TPW)])

    _edge_agg_loop(y_sp, z_sp, src_v, dst_v, upd_v, gsems, ssems)
    plsc.subcore_barrier()
    pltpu.sync_copy(z_sp.at[pl.ds(f0, 2 * _TPW)],
                    z_hbm.at[pl.ds(cid * 2 * _N_PAD + f0, 2 * _TPW)])


@functools.partial(
    pl.kernel,
    out_type=[
        jax.ShapeDtypeStruct((2 * _N_PAD,), jnp.float32),
        jax.ShapeDtypeStruct((4 * _N_PAD,), jnp.float32),
    ],
    mesh=_mesh,
    scratch_types=[
        pltpu.VMEM((_CH2, 128), jnp.int32),
        pltpu.VMEM((_CH2, 128), jnp.int32),
        pltpu.VMEM((8, 128), jnp.float32),
        pltpu.VMEM((_TPW,), jnp.float32),
        pltpu.VMEM((_TPW,), jnp.float32),
        pltpu.VMEM((_TPW,), jnp.float32),
        pltpu.VMEM((_TPW,), jnp.float32),
        pltpu.VMEM((_TPW,), jnp.float32),
        pltpu.VMEM((_TPW,), jnp.float32),
        pltpu.VMEM((_TPW,), jnp.float32),
        pltpu.VMEM((_TPW,), jnp.float32),
        pltpu.VMEM((_TPW,), jnp.float32),
        pltpu.VMEM((6, 16), jnp.float32),
        pltpu.VMEM_SHARED((2 * _N_PAD,), jnp.float32),
        pltpu.VMEM_SHARED((2 * _N_PAD,), jnp.float32),
    ] + [pltpu.SemaphoreType.DMA] * 16,
)
def _sc_layer2(deg_hbm, y1p_hbm, z1_hbm, consts_hbm, src2_hbm, dst2_hbm,
               zeros_hbm, y2_hbm, z_hbm,
               src_v, dst_v, upd_v,
               degb, y10b, y11b, za0b, za1b, zb0b, zb1b, yb0, yb1, cv,
               y_sp, z_sp, *sems):
    cid = lax.axis_index("c")
    sid = lax.axis_index("s")
    wid = sid * 2 + cid
    r0 = sid * _TPW
    f0 = sid * (2 * _TPW)
    gsems, ssems = sems[:8], sems[8:]

    pltpu.sync_copy(src2_hbm.at[wid], src_v)
    pltpu.sync_copy(dst2_hbm.at[wid], dst_v)
    pltpu.sync_copy(deg_hbm.at[pl.ds(r0, _TPW)], degb)
    pltpu.sync_copy(y1p_hbm.at[pl.ds(r0, _TPW)], y10b)
    pltpu.sync_copy(y1p_hbm.at[pl.ds(_N_PAD + r0, _TPW)], y11b)
    pltpu.sync_copy(z1_hbm.at[pl.ds(r0, _TPW)], za0b)
    pltpu.sync_copy(z1_hbm.at[pl.ds(_N_PAD + r0, _TPW)], za1b)
    pltpu.sync_copy(z1_hbm.at[pl.ds(2 * _N_PAD + r0, _TPW)], zb0b)
    pltpu.sync_copy(z1_hbm.at[pl.ds(3 * _N_PAD + r0, _TPW)], zb1b)
    pltpu.sync_copy(consts_hbm, cv)
    pltpu.sync_copy(zeros_hbm.at[pl.ds(f0, 2 * _TPW)],
                    z_sp.at[pl.ds(f0, 2 * _TPW)])

    g0 = cv[0, :]
    g1 = cv[1, :]
    bl0 = cv[2, :]
    bl1 = cv[3, :]
    b10 = cv[4, :]
    b11 = cv[5, :]
    for v in range(_TPW // 16):
        ix = pl.ds(v * 16, 16)
        dinv = _rsqrt16(degb[ix] + 1.0)
        o0 = dinv * (za0b[ix] + zb0b[ix] + y10b[ix]) + b10
        o1 = dinv * (za1b[ix] + zb1b[ix] + y11b[ix]) + b11
        d = (o0 - o1) * 0.5
        r = _rsqrt16(d * d + 1e-5)
        t = d * r
        a0 = jnp.maximum(t * g0 + bl0, 0.0)
        a1 = jnp.maximum(bl1 - t * g1, 0.0)
        node = r0 + v * 16 + lax.iota(jnp.int32, 16)
        live = node < _N
        yb0[ix] = jnp.where(live, a0 * dinv, 0.0)
        yb1[ix] = jnp.where(live, a1 * dinv, 0.0)
    pltpu.sync_copy(yb0, y_sp.at[pl.ds(r0, _TPW)])
    pltpu.sync_copy(yb1, y_sp.at[pl.ds(_N_PAD + r0, _TPW)])
    plsc.subcore_barrier()
    pltpu.sync_copy(y_sp.at[pl.ds(wid * _TPW, _TPW)],
                    y2_hbm.at[pl.ds(wid * _TPW, _TPW)])

    _edge_agg_loop(y_sp, z_sp, src_v, dst_v, upd_v, gsems, ssems)
    plsc.subcore_barrier()
    pltpu.sync_copy(z_sp.at[pl.ds(f0, 2 * _TPW)],
                    z_hbm.at[pl.ds(cid * 2 * _N_PAD + f0, 2 * _TPW)])


_BLK = 1000


def _tc1_body(x_ref, prm_ref, h_ref):
    prm = prm_ref[...]
    xb = x_ref[...]
    h0 = jnp.sum(xb * prm[0:1, :], axis=1, keepdims=True)
    h1 = jnp.sum(xb * prm[1:2, :], axis=1, keepdims=True)
    h_ref[...] = jnp.concatenate([h0, h1], axis=1)


_tc1 = pl.pallas_call(
    _tc1_body,
    grid=(_N // _BLK,),
    in_specs=[
        pl.BlockSpec((_BLK, _D), lambda i: (i, 0)),
        pl.BlockSpec((8, _D), lambda i: (0, 0)),
    ],
    out_specs=pl.BlockSpec((_BLK, _H), lambda i: (i, 0)),
    out_shape=jax.ShapeDtypeStruct((_N, _H), jnp.float32),
)


def _tc3_body(za0, za1, zb0, zb1, y20, y21, deg, prm_ref, sf, o_ref):
    prm = prm_ref[...]
    dinv = lax.rsqrt(deg[...] + 1.0)
    p0 = dinv * (za0[...] + zb0[...] + y20[...])
    p1 = dinv * (za1[...] + zb1[...] + y21[...])
    h = p0 * prm[2:3, :] + p1 * prm[3:4, :] + prm[4:5, :]
    o_ref[...] = jnp.maximum(h, 0.0) * sf[...]


_col = pl.BlockSpec((_BLK, 1), lambda i: (i, 0))
_tc3 = pl.pallas_call(
    _tc3_body,
    grid=(_N // _BLK,),
    in_specs=[_col, _col, _col, _col, _col, _col, _col,
              pl.BlockSpec((8, _D), lambda i: (0, 0)), _col],
    out_specs=pl.BlockSpec((_BLK, _D), lambda i: (i, 0)),
    out_shape=jax.ShapeDtypeStruct((_N, _D), jnp.float32),
)


def kernel(x, edge_index, size_factors, W1, b1, ln_g, ln_b, W2, b2):
    src, dst = edge_index[0], edge_index[1]

    padn = _E_PAD - _E
    pad_idx = _N + (jnp.arange(padn, dtype=jnp.int32) % 128)
    src_p = jnp.concatenate([src, pad_idx])
    dst_p = jnp.concatenate([dst, pad_idx])
    dsth = dst_p.reshape(16 * _CHH, 128)
    srcw = src_p.reshape(_NW, _EPT)
    dstw = dst_p.reshape(_NW, _EPT)
    src2 = jnp.concatenate([srcw, srcw + _N_PAD], axis=1).reshape(
        _NW, _CH2, 128)
    dst2 = jnp.concatenate([dstw, dstw + _N_PAD], axis=1).reshape(
        _NW, _CH2, 128)

    zeros = jnp.zeros((2 * _N_PAD,), jnp.float32)
    npad0 = jnp.zeros((_N_PAD - _N,), jnp.float32)

    def pad128(v):
        return jnp.zeros((_D,), jnp.float32).at[: v.shape[0]].set(v)

    prm = jnp.stack([
        W1[:, 0], W1[:, 1],
        W2[0], W2[1],
        b2,
        pad128(ln_g), pad128(ln_b), pad128(b1),
    ])
    consts = jnp.tile(
        jnp.stack([ln_g[0], ln_g[1], ln_b[0], ln_b[1], b1[0], b1[1]])[:, None],
        (1, 16)).astype(jnp.float32)

    h1 = _tc1(x, prm)
    h1p = jnp.concatenate([h1[:, 0], npad0, h1[:, 1], npad0])

    deg, y1p, z1 = _sc_layer1(h1p, dsth, src2, dst2, zeros)
    y2p, z2 = _sc_layer2(deg, y1p, z1, consts, src2, dst2, zeros)

    degc = deg[:_N][:, None]
    c = lambda a, o: a[o: o + _N][:, None]
    return _tc3(c(z2, 0), c(z2, _N_PAD), c(z2, 2 * _N_PAD), c(z2, 3 * _N_PAD),
                c(y2p, 0), c(y2p, _N_PAD), degc, prm, size_factors)


# speedup vs baseline: 1.8483x; 1.0264x over previous
import functools

import jax
import jax.numpy as jnp
from jax import lax
from jax.experimental import pallas as pl
from jax.experimental.pallas import tpu as pltpu
from jax.experimental.pallas import tpu_sc as plsc

_N = 10000
_D = 128
_H = 2
_E = 320000

_N_PAD = 10240
_NW = 32
_EPT = 10240
_E_PAD = _NW * _EPT
_CHH = _E_PAD // (128 * 16)
_CH2 = 2 * _EPT // 128
_TPW = _N_PAD // 16

_mesh = plsc.VectorSubcoreMesh(core_axis_name="c", subcore_axis_name="s")


def _rsqrt16(x):
    i = lax.bitcast_convert_type(x, jnp.int32)
    r = lax.bitcast_convert_type(jnp.int32(0x5F3759DF) - (i >> 1),
                                 jnp.float32)
    for _ in range(3):
        r = r * (1.5 - 0.5 * x * r * r)
    return r


def _edge_agg_loop(y_sp, z_sp, src_v, dst_v, upd_v, gsems, ssems):
    def fire_g(c, b):
        pltpu.async_copy(y_sp.at[src_v.at[c]], upd_v.at[b], gsems[b])

    def drain_g(b):
        pltpu.make_async_copy(y_sp.at[src_v.at[0]], upd_v.at[b],
                              gsems[b]).wait()

    def fire_s(c, b):
        pltpu.async_copy(upd_v.at[b], z_sp.at[dst_v.at[c]], ssems[b],
                         add=True)

    def drain_s(b):
        pltpu.make_async_copy(upd_v.at[b], z_sp.at[dst_v.at[0]],
                              ssems[b]).wait()

    for b in range(4):
        fire_g(b, b)
    for c in range(4):
        drain_g(c)
        fire_s(c, c)
        fire_g(c + 4, (c + 4) % 8)

    def body(i, carry):
        c0 = 4 + i * 8
        for k in range(8):
            c = c0 + k
            b = (4 + k) % 8
            drain_g(b)
            fire_s(c, b)
            drain_s((b + 4) % 8)
            fire_g(c + 4, (b + 4) % 8)
        return carry

    lax.fori_loop(0, (_CH2 - 8) // 8, body, 0)
    for k in range(4):
        c = _CH2 - 4 + k
        b = c % 8
        drain_g(b)
        fire_s(c, b)
        drain_s((b + 4) % 8)
    for k in range(4):
        drain_s((_CH2 - 4 + k) % 8)


@functools.partial(
    pl.kernel,
    out_type=[
        jax.ShapeDtypeStruct((_N_PAD,), jnp.float32),
        jax.ShapeDtypeStruct((2 * _N_PAD,), jnp.float32),
        jax.ShapeDtypeStruct((4 * _N_PAD,), jnp.float32),
    ],
    mesh=_mesh,
    scratch_types=[
        pltpu.VMEM((_CHH, 128), jnp.int32),
        pltpu.VMEM((_CH2, 128), jnp.int32),
        pltpu.VMEM((_CH2, 128), jnp.int32),
        pltpu.VMEM((128,), jnp.float32),
        pltpu.VMEM((8, 128), jnp.float32),
        pltpu.VMEM((2, 64, _D), jnp.float32),
        pltpu.VMEM((2, _D), jnp.float32),
        pltpu.VMEM((_TPW,), jnp.float32),
        pltpu.VMEM((_TPW,), jnp.float32),
        pltpu.VMEM((_TPW,), jnp.float32),
        pltpu.VMEM((_TPW,), jnp.float32),
        pltpu.VMEM((_TPW,), jnp.float32),
        pltpu.VMEM_SHARED((_N_PAD,), jnp.float32),
        pltpu.VMEM_SHARED((2 * _N_PAD,), jnp.float32),
        pltpu.VMEM_SHARED((2 * _N_PAD,), jnp.float32),
    ] + [pltpu.SemaphoreType.DMA] * 19,
)
def _sc_layer1(x_hbm, w1c_hbm, dsth_hbm, src2_hbm, dst2_hbm, zeros_hbm,
               deg_hbm, y1_hbm, z_hbm,
               dsth_v, src_v, dst_v, ones_v, upd_v, xb, w1v,
               degb, h0b, h1b, yb0, yb1,
               cnt_sp, y_sp, z_sp, *sems):
    cid = lax.axis_index("c")
    sid = lax.axis_index("s")
    wid = sid * 2 + cid
    r0 = sid * _TPW
    f0 = sid * (2 * _TPW)
    gsems, ssems, hsem = sems[:8], sems[8:16], sems[16]
    xsems = sems[17:19]

    def stage_x(bb):
        pltpu.async_copy(x_hbm.at[pl.ds(r0 + bb * 64, 64)], xb.at[bb % 2],
                         xsems[bb % 2])

    def wait_x(bb):
        pltpu.make_async_copy(x_hbm.at[pl.ds(r0, 64)], xb.at[bb % 2],
                              xsems[bb % 2]).wait()

    stage_x(0)
    pltpu.sync_copy(zeros_hbm.at[pl.ds(r0, _TPW)], cnt_sp.at[pl.ds(r0, _TPW)])
    pltpu.sync_copy(dsth_hbm.at[pl.ds(sid * _CHH, _CHH)], dsth_v)
    for k in range(8):
        ones_v[pl.ds(k * 16, 16)] = jnp.ones((16,), jnp.float32)
    plsc.subcore_barrier()

    def fire_h(j, carry):
        pltpu.async_copy(ones_v, cnt_sp.at[dsth_v.at[j]], hsem, add=True)
        return carry

    def drain_h(j, carry):
        pltpu.make_async_copy(ones_v, cnt_sp.at[dsth_v.at[0]], hsem).wait()
        return carry

    lax.fori_loop(0, _CHH, fire_h, 0)
    pltpu.sync_copy(src2_hbm.at[wid], src_v)
    pltpu.sync_copy(dst2_hbm.at[wid], dst_v)
    pltpu.sync_copy(w1c_hbm, w1v)
    pltpu.sync_copy(zeros_hbm.at[pl.ds(f0, 2 * _TPW)],
                    z_sp.at[pl.ds(f0, 2 * _TPW)])
    w0 = [w1v[0, pl.ds(k * 16, 16)] for k in range(8)]
    w1 = [w1v[1, pl.ds(k * 16, 16)] for k in range(8)]
    ilane = lax.iota(jnp.int32, 16)
    bfly = [ilane ^ sh for sh in (8, 4, 2, 1)]

    def _allsum(v):
        for p in bfly:
            v = v + jnp.take(v, p)
        return v

    for bb in range(_TPW // 64):
        wait_x(bb)
        if bb + 1 < _TPW // 64:
            stage_x(bb + 1)
        slot = bb % 2

        def groupbody(g, carry, slot=slot, bb=bb):
            hv0 = jnp.zeros((16,), jnp.float32)
            hv1 = jnp.zeros((16,), jnp.float32)
            for i in range(16):
                row = g * 16 + i
                xv = xb[slot, row, pl.ds(0, 16)]
                a0 = xv * w0[0]
                a1 = xv * w1[0]
                for k in range(1, 8):
                    xv = xb[slot, row, pl.ds(k * 16, 16)]
                    a0 = a0 + xv * w0[k]
                    a1 = a1 + xv * w1[k]
                hv0 = jnp.where(ilane == i, _allsum(a0), hv0)
                hv1 = jnp.where(ilane == i, _allsum(a1), hv1)
            h0b[pl.ds(bb * 64 + g * 16, 16)] = hv0
            h1b[pl.ds(bb * 64 + g * 16, 16)] = hv1
            return carry

        lax.fori_loop(0, 4, groupbody, 0)
    lax.fori_loop(0, _CHH, drain_h, 0)
    plsc.subcore_barrier()

    pltpu.sync_copy(cnt_sp.at[pl.ds(r0, _TPW)], degb)
    for v in range(_TPW // 16):
        ix = pl.ds(v * 16, 16)
        dinv = _rsqrt16(degb[ix] + 1.0)
        yb0[ix] = h0b[ix] * dinv
        yb1[ix] = h1b[ix] * dinv
    pltpu.sync_copy(yb0, y_sp.at[pl.ds(r0, _TPW)])
    pltpu.sync_copy(yb1, y_sp.at[pl.ds(_N_PAD + r0, _TPW)])
    @pl.when(cid == 0)
    def _():
        pltpu.sync_copy(cnt_sp.at[pl.ds(r0, _TPW)],
                        deg_hbm.at[pl.ds(r0, _TPW)])
    plsc.subcore_barrier()
    pltpu.sync_copy(y_sp.at[pl.ds(wid * _TPW, _TPW)],
                    y1_hbm.at[pl.ds(wid * _TPW, _TPW)])

    _edge_agg_loop(y_sp, z_sp, src_v, dst_v, upd_v, gsems, ssems)
    plsc.subcore_barrier()
    pltpu.sync_copy(z_sp.at[pl.ds(f0, 2 * _TPW)],
                    z_hbm.at[pl.ds(cid * 2 * _N_PAD + f0, 2 * _TPW)])


@functools.partial(
    pl.kernel,
    out_type=[
        jax.ShapeDtypeStruct((2 * _N_PAD,), jnp.float32),
        jax.ShapeDtypeStruct((4 * _N_PAD,), jnp.float32),
    ],
    mesh=_mesh,
    scratch_types=[
        pltpu.VMEM((_CH2, 128), jnp.int32),
        pltpu.VMEM((_CH2, 128), jnp.int32),
        pltpu.VMEM((8, 128), jnp.float32),
        pltpu.VMEM((_TPW,), jnp.float32),
        pltpu.VMEM((_TPW,), jnp.float32),
        pltpu.VMEM((_TPW,), jnp.float32),
        pltpu.VMEM((_TPW,), jnp.float32),
        pltpu.VMEM((_TPW,), jnp.float32),
        pltpu.VMEM((_TPW,), jnp.float32),
        pltpu.VMEM((_TPW,), jnp.float32),
        pltpu.VMEM((_TPW,), jnp.float32),
        pltpu.VMEM((_TPW,), jnp.float32),
        pltpu.VMEM((6, 16), jnp.float32),
        pltpu.VMEM_SHARED((2 * _N_PAD,), jnp.float32),
        pltpu.VMEM_SHARED((2 * _N_PAD,), jnp.float32),
    ] + [pltpu.SemaphoreType.DMA] * 16,
)
def _sc_layer2(deg_hbm, y1p_hbm, z1_hbm, consts_hbm, src2_hbm, dst2_hbm,
               zeros_hbm, y2_hbm, z_hbm,
               src_v, dst_v, upd_v,
               degb, y10b, y11b, za0b, za1b, zb0b, zb1b, yb0, yb1, cv,
               y_sp, z_sp, *sems):
    cid = lax.axis_index("c")
    sid = lax.axis_index("s")
    wid = sid * 2 + cid
    r0 = sid * _TPW
    f0 = sid * (2 * _TPW)
    gsems, ssems = sems[:8], sems[8:]

    pltpu.sync_copy(src2_hbm.at[wid], src_v)
    pltpu.sync_copy(dst2_hbm.at[wid], dst_v)
    pltpu.sync_copy(deg_hbm.at[pl.ds(r0, _TPW)], degb)
    pltpu.sync_copy(y1p_hbm.at[pl.ds(r0, _TPW)], y10b)
    pltpu.sync_copy(y1p_hbm.at[pl.ds(_N_PAD + r0, _TPW)], y11b)
    pltpu.sync_copy(z1_hbm.at[pl.ds(r0, _TPW)], za0b)
    pltpu.sync_copy(z1_hbm.at[pl.ds(_N_PAD + r0, _TPW)], za1b)
    pltpu.sync_copy(z1_hbm.at[pl.ds(2 * _N_PAD + r0, _TPW)], zb0b)
    pltpu.sync_copy(z1_hbm.at[pl.ds(3 * _N_PAD + r0, _TPW)], zb1b)
    pltpu.sync_copy(consts_hbm, cv)
    pltpu.sync_copy(zeros_hbm.at[pl.ds(f0, 2 * _TPW)],
                    z_sp.at[pl.ds(f0, 2 * _TPW)])

    g0 = cv[0, :]
    g1 = cv[1, :]
    bl0 = cv[2, :]
    bl1 = cv[3, :]
    b10 = cv[4, :]
    b11 = cv[5, :]
    for v in range(_TPW // 16):
        ix = pl.ds(v * 16, 16)
        dinv = _rsqrt16(degb[ix] + 1.0)
        o0 = dinv * (za0b[ix] + zb0b[ix] + y10b[ix]) + b10
        o1 = dinv * (za1b[ix] + zb1b[ix] + y11b[ix]) + b11
        d = (o0 - o1) * 0.5
        r = _rsqrt16(d * d + 1e-5)
        t = d * r
        a0 = jnp.maximum(t * g0 + bl0, 0.0)
        a1 = jnp.maximum(bl1 - t * g1, 0.0)
        node = r0 + v * 16 + lax.iota(jnp.int32, 16)
        live = node < _N
        yb0[ix] = jnp.where(live, a0 * dinv, 0.0)
        yb1[ix] = jnp.where(live, a1 * dinv, 0.0)
    pltpu.sync_copy(yb0, y_sp.at[pl.ds(r0, _TPW)])
    pltpu.sync_copy(yb1, y_sp.at[pl.ds(_N_PAD + r0, _TPW)])
    plsc.subcore_barrier()
    pltpu.sync_copy(y_sp.at[pl.ds(wid * _TPW, _TPW)],
                    y2_hbm.at[pl.ds(wid * _TPW, _TPW)])

    _edge_agg_loop(y_sp, z_sp, src_v, dst_v, upd_v, gsems, ssems)
    plsc.subcore_barrier()
    pltpu.sync_copy(z_sp.at[pl.ds(f0, 2 * _TPW)],
                    z_hbm.at[pl.ds(cid * 2 * _N_PAD + f0, 2 * _TPW)])


_BLK = 1000


def _tc3_body(za0, za1, zb0, zb1, y20, y21, deg, prm_ref, sf, o_ref):
    prm = prm_ref[...]
    dinv = lax.rsqrt(deg[...] + 1.0)
    p0 = dinv * (za0[...] + zb0[...] + y20[...])
    p1 = dinv * (za1[...] + zb1[...] + y21[...])
    h = p0 * prm[2:3, :] + p1 * prm[3:4, :] + prm[4:5, :]
    o_ref[...] = jnp.maximum(h, 0.0) * sf[...]


_col = pl.BlockSpec((_BLK, 1), lambda i: (i, 0))
_tc3 = pl.pallas_call(
    _tc3_body,
    grid=(_N // _BLK,),
    in_specs=[_col, _col, _col, _col, _col, _col, _col,
              pl.BlockSpec((8, _D), lambda i: (0, 0)), _col],
    out_specs=pl.BlockSpec((_BLK, _D), lambda i: (i, 0)),
    out_shape=jax.ShapeDtypeStruct((_N, _D), jnp.float32),
)


def kernel(x, edge_index, size_factors, W1, b1, ln_g, ln_b, W2, b2):
    src, dst = edge_index[0], edge_index[1]

    padn = _E_PAD - _E
    pad_idx = _N + (jnp.arange(padn, dtype=jnp.int32) % 128)
    src_p = jnp.concatenate([src, pad_idx])
    dst_p = jnp.concatenate([dst, pad_idx])
    dsth = dst_p.reshape(16 * _CHH, 128)
    srcw = src_p.reshape(_NW, _EPT)
    dstw = dst_p.reshape(_NW, _EPT)
    src2 = jnp.concatenate([srcw, srcw + _N_PAD], axis=1).reshape(
        _NW, _CH2, 128)
    dst2 = jnp.concatenate([dstw, dstw + _N_PAD], axis=1).reshape(
        _NW, _CH2, 128)

    zeros = jnp.zeros((2 * _N_PAD,), jnp.float32)
    npad0 = jnp.zeros((_N_PAD - _N,), jnp.float32)

    def pad128(v):
        return jnp.zeros((_D,), jnp.float32).at[: v.shape[0]].set(v)

    prm = jnp.stack([
        W1[:, 0], W1[:, 1],
        W2[0], W2[1],
        b2,
        pad128(ln_g), pad128(ln_b), pad128(b1),
    ])
    consts = jnp.tile(
        jnp.stack([ln_g[0], ln_g[1], ln_b[0], ln_b[1], b1[0], b1[1]])[:, None],
        (1, 16)).astype(jnp.float32)

    xp = jnp.concatenate([x, jnp.zeros((_N_PAD - _N, _D), jnp.float32)])
    w1c = jnp.stack([W1[:, 0], W1[:, 1]])

    deg, y1p, z1 = _sc_layer1(xp, w1c, dsth, src2, dst2, zeros)
    y2p, z2 = _sc_layer2(deg, y1p, z1, consts, src2, dst2, zeros)

    degc = deg[:_N][:, None]
    c = lambda a, o: a[o: o + _N][:, None]
    return _tc3(c(z2, 0), c(z2, _N_PAD), c(z2, 2 * _N_PAD), c(z2, 3 * _N_PAD),
                c(y2p, 0), c(y2p, _N_PAD), degc, prm, size_factors)
